# baseline (speedup 1.0000x reference)
import functools

import jax
import jax.numpy as jnp
from jax import lax
from jax.experimental import pallas as pl
from jax.experimental.pallas import tpu as pltpu
from jax.experimental.pallas import tpu_sc as plsc

NUM_CORES = 2
NUM_SUBCORES = 16
NUM_WORKERS = NUM_CORES * NUM_SUBCORES
CHUNK = 128
LANES = 16


def kernel(token_ids, weights):
    B, S = token_ids.shape
    V, D = weights.shape
    n = B * S
    assert n % (NUM_WORKERS * CHUNK) == 0 and V % 2 == 0 and D == 64
    chunks_per_w = n // (NUM_WORKERS * CHUNK)
    idx = token_ids.reshape(NUM_WORKERS, chunks_per_w, CHUNK)
    table2 = weights.reshape(V // 2, 2 * D)

    mesh = plsc.VectorSubcoreMesh(core_axis_name="c", subcore_axis_name="s")

    @functools.partial(
        pl.kernel,
        mesh=mesh,
        out_type=jax.ShapeDtypeStruct((n, D), jnp.float32),
        scratch_types=[
            pltpu.VMEM((chunks_per_w, CHUNK), jnp.int32),
            pltpu.VMEM((CHUNK,), jnp.int32),
            pltpu.VMEM((CHUNK, 2 * D), jnp.float32),
            pltpu.VMEM((CHUNK, D), jnp.float32),
            pltpu.SemaphoreType.DMA,
        ],
    )
    def gather_kernel(table_hbm, idx_hbm, out_hbm, idx_v, g_v, rows_v, out_v,
                      sem):
        wid = lax.axis_index("s") * NUM_CORES + lax.axis_index("c")
        base = wid * (chunks_per_w * CHUNK)
        pltpu.sync_copy(idx_hbm.at[wid], idx_v)

        @pl.loop(0, chunks_per_w)
        def _(j):
            for c in range(CHUNK // LANES):
                sl = pl.ds(c * LANES, LANES)
                g_v[sl] = jax.lax.shift_right_logical(idx_v[j, sl], 1)
            pltpu.async_copy(table_hbm.at[g_v], rows_v, sem).wait()

            @pl.loop(0, CHUNK, step=LANES)
            def _(r0):
                offs = (idx_v[j, pl.ds(r0, LANES)] & 1) * D
                for t in range(LANES):
                    off = offs[t]
                    for c in range(D // LANES):
                        out_v[r0 + t, pl.ds(c * LANES, LANES)] = rows_v[
                            r0 + t, pl.ds(off + c * LANES, LANES)
                        ]

            pltpu.sync_copy(out_v, out_hbm.at[pl.ds(base + j * CHUNK, CHUNK)])

    out = gather_kernel(table2, idx)
    return out.reshape(B, S, D)


# speedup vs baseline: 1.3564x; 1.3564x over previous
import functools

import jax
import jax.numpy as jnp
from jax import lax
from jax.experimental import pallas as pl
from jax.experimental.pallas import tpu as pltpu
from jax.experimental.pallas import tpu_sc as plsc

NUM_CORES = 2
NUM_SUBCORES = 16
NUM_WORKERS = NUM_CORES * NUM_SUBCORES
CHUNK = 128
LANES = 16


def kernel(token_ids, weights):
    B, S = token_ids.shape
    V, D = weights.shape
    n = B * S
    assert n % (NUM_WORKERS * 2 * CHUNK) == 0 and V % 2 == 0 and D == 64
    chunks_per_w = n // (NUM_WORKERS * CHUNK)
    idx = token_ids.reshape(NUM_WORKERS, chunks_per_w, CHUNK)
    table2 = weights.reshape(V // 2, 2 * D)

    mesh = plsc.VectorSubcoreMesh(core_axis_name="c", subcore_axis_name="s")

    @functools.partial(
        pl.kernel,
        mesh=mesh,
        out_type=jax.ShapeDtypeStruct((n, D), jnp.float32),
        scratch_types=[
            pltpu.VMEM((chunks_per_w, CHUNK), jnp.int32),
            pltpu.VMEM((2, CHUNK), jnp.int32),
            pltpu.VMEM((2, CHUNK, 2 * D), jnp.float32),
            pltpu.VMEM((2, CHUNK, D), jnp.float32),
            pltpu.SemaphoreType.DMA((2,)),
            pltpu.SemaphoreType.DMA((2,)),
        ],
    )
    def gather_kernel(table_hbm, idx_hbm, out_hbm, idx_v, g_v, rows_v, out_v,
                      gsem, osem):
        wid = lax.axis_index("s") * NUM_CORES + lax.axis_index("c")
        base = wid * (chunks_per_w * CHUNK)
        pltpu.sync_copy(idx_hbm.at[wid], idx_v)

        def start_gather(j, b):
            for c in range(CHUNK // LANES):
                sl = pl.ds(c * LANES, LANES)
                g_v[b, sl] = jax.lax.shift_right_logical(idx_v[j, sl], 1)
            pltpu.async_copy(table_hbm.at[g_v.at[b]], rows_v.at[b],
                             gsem.at[b])

        start_gather(0, 0)
        start_gather(1, 1)

        @pl.loop(0, chunks_per_w, step=2)
        def _(j):
            for b in range(2):
                jj = j + b
                pltpu.make_async_copy(table_hbm.at[g_v.at[b]], rows_v.at[b],
                                      gsem.at[b]).wait()
                @pl.when(jj >= 2)
                def _():
                    pltpu.make_async_copy(
                        out_v.at[b], out_hbm.at[pl.ds(base, CHUNK)],
                        osem.at[b]).wait()

                @pl.loop(0, CHUNK, step=LANES)
                def _(r0):
                    offs = (idx_v[jj, pl.ds(r0, LANES)] & 1) * D
                    for t in range(LANES):
                        off = offs[t]
                        for c in range(D // LANES):
                            out_v[b, r0 + t, pl.ds(c * LANES, LANES)] = (
                                rows_v[b, r0 + t,
                                       pl.ds(off + c * LANES, LANES)]
                            )

                pltpu.async_copy(
                    out_v.at[b],
                    out_hbm.at[pl.ds(base + jj * CHUNK, CHUNK)],
                    osem.at[b])

                @pl.when(jj + 2 < chunks_per_w)
                def _():
                    start_gather(jj + 2, b)

        for b in range(2):
            pltpu.make_async_copy(out_v.at[b],
                                  out_hbm.at[pl.ds(base, CHUNK)],
                                  osem.at[b]).wait()

    out = gather_kernel(table2, idx)
    return out.reshape(B, S, D)


# speedup vs baseline: 1.4754x; 1.0877x over previous
import functools

import jax
import jax.numpy as jnp
from jax import lax
from jax.experimental import pallas as pl
from jax.experimental.pallas import tpu as pltpu
from jax.experimental.pallas import tpu_sc as plsc

NUM_CORES = 2
NUM_SUBCORES = 16
NUM_WORKERS = NUM_CORES * NUM_SUBCORES
CHUNK = 128
LANES = 16
NBUF = 4


def kernel(token_ids, weights):
    B, S = token_ids.shape
    V, D = weights.shape
    n = B * S
    assert n % (NUM_WORKERS * 2 * CHUNK) == 0 and V % 2 == 0 and D == 64
    chunks_per_w = n // (NUM_WORKERS * CHUNK)
    idx = token_ids.reshape(NUM_WORKERS, chunks_per_w, CHUNK)
    table2 = weights.reshape(V // 2, 2 * D)

    mesh = plsc.VectorSubcoreMesh(core_axis_name="c", subcore_axis_name="s")

    @functools.partial(
        pl.kernel,
        mesh=mesh,
        out_type=jax.ShapeDtypeStruct((n, 2 * D), jnp.float32),
        scratch_types=[
            pltpu.VMEM((chunks_per_w, CHUNK), jnp.int32),
            pltpu.VMEM((NBUF, CHUNK), jnp.int32),
            pltpu.VMEM((NBUF, CHUNK, 2 * D), jnp.float32),
            pltpu.SemaphoreType.DMA((NBUF,)),
            pltpu.SemaphoreType.DMA((NBUF,)),
        ],
    )
    def gather_kernel(table_hbm, idx_hbm, out_hbm, idx_v, g_v, rows_v,
                      gsem, osem):
        wid = lax.axis_index("s") * NUM_CORES + lax.axis_index("c")
        base = wid * (chunks_per_w * CHUNK)

        def start_gather(j, b):
            for c in range(CHUNK // LANES):
                sl = pl.ds(c * LANES, LANES)
                g_v[b, sl] = jax.lax.shift_right_logical(idx_v[j, sl], 1)
            pltpu.async_copy(table_hbm.at[g_v.at[b]], rows_v.at[b],
                             gsem.at[b])

        pltpu.sync_copy(idx_hbm.at[wid], idx_v)
        start_gather(0, 0)
        start_gather(1, 1)

        @pl.loop(0, chunks_per_w, step=NBUF)
        def _(j):
            for b in range(NBUF):
                jj = j + b
                pltpu.make_async_copy(table_hbm.at[g_v.at[b]], rows_v.at[b],
                                      gsem.at[b]).wait()

                b2 = (jj + 2) % NBUF
                @pl.when(jj >= 2)
                def _():
                    pltpu.make_async_copy(
                        rows_v.at[b2], out_hbm.at[pl.ds(base, CHUNK)],
                        osem.at[b2]).wait()
                @pl.when(jj + 2 < chunks_per_w)
                def _():
                    start_gather(jj + 2, b2)

                @pl.loop(0, CHUNK, step=LANES)
                def _(r0):
                    offs = (idx_v[jj, pl.ds(r0, LANES)] & 1) * D
                    for t in range(LANES):
                        @pl.when(offs[t] > 0)
                        def _():
                            for c in range(D // LANES):
                                rows_v[b, r0 + t, pl.ds(c * LANES, LANES)] = (
                                    rows_v[b, r0 + t,
                                           pl.ds(D + c * LANES, LANES)]
                                )

                pltpu.async_copy(
                    rows_v.at[b],
                    out_hbm.at[pl.ds(base + jj * CHUNK, CHUNK)],
                    osem.at[b])

        for jj in (chunks_per_w - 2, chunks_per_w - 1):
            pltpu.make_async_copy(rows_v.at[jj % NBUF],
                                  out_hbm.at[pl.ds(base, CHUNK)],
                                  osem.at[jj % NBUF]).wait()

    out2 = gather_kernel(table2, idx)
    return out2.reshape(B, S, 2 * D)[:, :, :D]
